# Initial kernel scaffold; baseline (speedup 1.0000x reference)
import jax
import jax.numpy as jnp
from jax.experimental import pallas as pl


def kernel(x, tables, W, b):
    raise NotImplementedError("write your pallas kernel here")


# speedup vs baseline: 2.4210x; 2.4210x over previous
import functools

import jax
import jax.numpy as jnp
from jax import lax
from jax.experimental import pallas as pl
from jax.experimental.pallas import tpu as pltpu
from jax.experimental.pallas import tpu_sc as plsc

B, T, F, D = 4096, 50, 26, 32
V = 100000
NF = 13
N_TOK = B * T

_info = plsc.get_sparse_core_info()
NC, NS = _info.num_cores, _info.num_subcores
NW = NC * NS
TOK_PER_W = N_TOK // NW
C = 128
CHUNKS = TOK_PER_W // C


def _sc_body(x_hbm, tab_hbm, w_hbm, b_hbm, out_hbm,
             x_v, idx_v, rows_cat, rows_con, w_v, b_v, sem_g, sem_o):
    wid = lax.axis_index("s") * NC + lax.axis_index("c")
    pltpu.sync_copy(w_hbm, w_v)
    pltpu.sync_copy(b_hbm, b_v)
    iota = lax.iota(jnp.int32, 16)
    iota_f = iota * F

    def chunk(k, carry):
        base = wid * TOK_PER_W + k * C
        pltpu.sync_copy(x_hbm.at[pl.ds(base * F, C * F)], x_v)

        for j in range(NF):
            for g in range(C // 16):
                xg = plsc.load_gather(x_v, [iota_f + (g * 16 * F + 2 * j)])
                idx_v[j, pl.ds(g * 16, 16)] = xg.astype(jnp.int32) + j * V
        gd = [pltpu.async_copy(tab_hbm.at[idx_v.at[j]], rows_cat.at[j], sem_g)
              for j in range(NF)]

        for j in range(NF):
            w0 = w_v[j, pl.ds(0, 16)]
            w1 = w_v[j, pl.ds(16, 16)]
            b0 = b_v[j, pl.ds(0, 16)]
            b1 = b_v[j, pl.ds(16, 16)]
            def tok(c, _, w0=w0, w1=w1, b0=b0, b1=b1, j=j):
                xc = plsc.load_gather(
                    x_v, [jnp.full((16,), c * F + 2 * j + 1, jnp.int32)])
                v0 = xc * w0 + b0
                v1 = xc * w1 + b1
                s0 = v0 / (1.0 + jnp.exp(-v0))
                s1 = v1 / (1.0 + jnp.exp(-v1))
                rows_con[j, c, pl.ds(0, 16)] = s0
                rows_con[j, c, pl.ds(16, 16)] = s1
                return _

            lax.fori_loop(0, C, tok, 0)

        od = [pltpu.async_copy(rows_con.at[j],
                               out_hbm.at[pl.ds(base, C), 2 * j + 1], sem_o)
              for j in range(NF)]
        for d in gd:
            d.wait()
        od += [pltpu.async_copy(rows_cat.at[j],
                                out_hbm.at[pl.ds(base, C), 2 * j], sem_o)
               for j in range(NF)]
        for d in od:
            d.wait()
        return carry

    lax.fori_loop(0, CHUNKS, chunk, 0)


@jax.jit
def kernel(x, tables, W, b):
    xr = x.reshape(N_TOK * F)
    tab = tables.reshape(NF * V, D)
    run = pl.kernel(
        _sc_body,
        out_type=jax.ShapeDtypeStruct((N_TOK, F, D), jnp.float32),
        mesh=plsc.VectorSubcoreMesh(core_axis_name="c", subcore_axis_name="s"),
        compiler_params=pltpu.CompilerParams(
            needs_layout_passes=False, use_tc_tiling_on_sc=False),
        scratch_types=[
            pltpu.VMEM((C * F,), jnp.float32),
            pltpu.VMEM((NF, C), jnp.int32),
            pltpu.VMEM((NF, C, D), jnp.float32),
            pltpu.VMEM((NF, C, D), jnp.float32),
            pltpu.VMEM((NF, D), jnp.float32),
            pltpu.VMEM((NF, D), jnp.float32),
            pltpu.SemaphoreType.DMA,
            pltpu.SemaphoreType.DMA,
        ],
    )
    out = run(xr, tab, W, b)
    return out.reshape(B, T, F, D)
